# baseline (speedup 1.0000x reference)
import jax
import jax.numpy as jnp
from jax.experimental import pallas as pl
from jax.experimental.pallas import tpu as pltpu

_C = 16


def _pool_body(s_ref, x_ref, p_ref, o_ref):
    v = x_ref[...]
    c, h, w = v.shape
    wp = jnp.dot(v.reshape(c * h, w), p_ref[...],
                 preferred_element_type=jnp.float32)
    wr = wp.reshape(c, h // 2, 2, w // 2)
    hp = wr[:, :, 0, :] + wr[:, :, 1, :]
    o_ref[...] = s_ref[0] * hp + s_ref[1]


def kernel(x, coefficient, bias):
    b, c, h, w = x.shape
    oh, ow = h // 2, w // 2
    xf = x.reshape(b * c, h, w)
    scale = jnp.concatenate([coefficient, bias])
    pair = jnp.repeat(jnp.eye(w // 2, dtype=x.dtype), 2, axis=0)
    out = pl.pallas_call(
        _pool_body,
        grid=(b * c // _C,),
        in_specs=[
            pl.BlockSpec(memory_space=pltpu.SMEM),
            pl.BlockSpec((_C, h, w), lambda i: (i, 0, 0)),
            pl.BlockSpec((w, w // 2), lambda i: (0, 0)),
        ],
        out_specs=pl.BlockSpec((_C, oh, ow), lambda i: (i, 0, 0)),
        out_shape=jax.ShapeDtypeStruct((b * c, oh, ow), x.dtype),
        compiler_params=pltpu.CompilerParams(
            dimension_semantics=("parallel",),
            vmem_limit_bytes=48 * 1024 * 1024,
        ),
    )(scale, xf, pair)
    return out.reshape(b, c, oh, ow)
